# baseline (device time: 117909 ns/iter reference)
import jax
import jax.numpy as jnp
from jax import lax
from jax.experimental import pallas as pl
from jax.experimental.pallas import tpu as pltpu

N_CHUNKS = 16


def kernel(x):
    m, n = x.shape
    half = m // 2
    rows = half // N_CHUNKS

    def body(x_hbm, out_hbm, xv_ref, send_ref, recv_ref, mine_ref, oth_ref,
             load_sems, st_my_sems, st_oth_sems, s_sem1, r_sem1, s_sem2,
             r_sem2):
        my_x = lax.axis_index("x")
        my_y = lax.axis_index("y")
        y_nbr = (my_x, 1 - my_y)
        x_nbr = (1 - my_x, my_y)

        off = my_x * half
        off_oth = (1 - my_x) * half

        loads = []
        for c in range(N_CHUNKS):
            cp = pltpu.make_async_copy(
                x_hbm.at[pl.ds(off + c * rows, rows)],
                xv_ref.at[pl.ds(c * rows, rows)],
                load_sems.at[c],
            )
            cp.start()
            loads.append(cp)

        barrier = pltpu.get_barrier_semaphore()
        for nbr in (y_nbr, x_nbr):
            pl.semaphore_signal(
                barrier, inc=1, device_id=nbr,
                device_id_type=pl.DeviceIdType.MESH,
            )
        pl.semaphore_wait(barrier, 2)

        phase1 = []
        for c in range(N_CHUNKS):
            csl = pl.ds(c * rows, rows)
            loads[c].wait()
            send_ref[csl, :] = xv_ref[csl, :].astype(jnp.bfloat16)
            rdma = pltpu.make_async_remote_copy(
                src_ref=send_ref.at[csl],
                dst_ref=recv_ref.at[csl],
                send_sem=s_sem1.at[c],
                recv_sem=r_sem1.at[c],
                device_id=y_nbr,
                device_id_type=pl.DeviceIdType.MESH,
            )
            rdma.start()
            phase1.append(rdma)

        phase2 = []
        stores = []
        for c in range(N_CHUNKS):
            phase1[c].wait_recv()
            csl = pl.ds(c * rows, rows)
            mine_ref[csl, :] = send_ref[csl, :] + recv_ref[csl, :]
            rdma = pltpu.make_async_remote_copy(
                src_ref=mine_ref.at[csl],
                dst_ref=oth_ref.at[csl],
                send_sem=s_sem2.at[c],
                recv_sem=r_sem2.at[c],
                device_id=x_nbr,
                device_id_type=pl.DeviceIdType.MESH,
            )
            rdma.start()
            phase2.append(rdma)
            st = pltpu.make_async_copy(
                mine_ref.at[csl],
                out_hbm.at[pl.ds(off + c * rows, rows)],
                st_my_sems.at[c],
            )
            st.start()
            stores.append(st)

        for c in range(N_CHUNKS):
            phase2[c].wait_recv()
            csl = pl.ds(c * rows, rows)
            st = pltpu.make_async_copy(
                oth_ref.at[csl],
                out_hbm.at[pl.ds(off_oth + c * rows, rows)],
                st_oth_sems.at[c],
            )
            st.start()
            stores.append(st)

        for st in stores:
            st.wait()
        for c in range(N_CHUNKS):
            phase1[c].wait_send()
            phase2[c].wait_send()

    return pl.pallas_call(
        body,
        out_shape=jax.ShapeDtypeStruct((m, n), jnp.bfloat16),
        in_specs=[pl.BlockSpec(memory_space=pltpu.MemorySpace.HBM)],
        out_specs=pl.BlockSpec(memory_space=pltpu.MemorySpace.HBM),
        scratch_shapes=[
            pltpu.VMEM((half, n), jnp.float32),
            pltpu.VMEM((half, n), jnp.bfloat16),
            pltpu.VMEM((half, n), jnp.bfloat16),
            pltpu.VMEM((half, n), jnp.bfloat16),
            pltpu.VMEM((half, n), jnp.bfloat16),
            pltpu.SemaphoreType.DMA((N_CHUNKS,)),
            pltpu.SemaphoreType.DMA((N_CHUNKS,)),
            pltpu.SemaphoreType.DMA((N_CHUNKS,)),
            pltpu.SemaphoreType.DMA((N_CHUNKS,)),
            pltpu.SemaphoreType.DMA((N_CHUNKS,)),
            pltpu.SemaphoreType.DMA((N_CHUNKS,)),
            pltpu.SemaphoreType.DMA((N_CHUNKS,)),
        ],
        compiler_params=pltpu.CompilerParams(
            collective_id=0, vmem_limit_bytes=100 * 1024 * 1024
        ),
    )(x)


# device time: 115788 ns/iter; 1.0183x vs baseline; 1.0183x over previous
import jax
import jax.numpy as jnp
from jax import lax
from jax.experimental import pallas as pl
from jax.experimental.pallas import tpu as pltpu

N_CHUNKS = 32


def kernel(x):
    m, n = x.shape
    half = m // 2
    rows = half // N_CHUNKS

    def body(x_hbm, out_hbm, xv_ref, send_ref, recv_ref, mine_ref, oth_ref,
             load_sems, st_my_sems, st_oth_sems, s_sem1, r_sem1, s_sem2,
             r_sem2):
        my_x = lax.axis_index("x")
        my_y = lax.axis_index("y")
        y_nbr = (my_x, 1 - my_y)
        x_nbr = (1 - my_x, my_y)

        off = my_x * half
        off_oth = (1 - my_x) * half

        loads = []
        for c in range(N_CHUNKS):
            cp = pltpu.make_async_copy(
                x_hbm.at[pl.ds(off + c * rows, rows)],
                xv_ref.at[pl.ds(c * rows, rows)],
                load_sems.at[c],
            )
            cp.start()
            loads.append(cp)

        barrier = pltpu.get_barrier_semaphore()
        for nbr in (y_nbr, x_nbr):
            pl.semaphore_signal(
                barrier, inc=1, device_id=nbr,
                device_id_type=pl.DeviceIdType.MESH,
            )
        pl.semaphore_wait(barrier, 2)

        phase1 = []
        for c in range(N_CHUNKS):
            csl = pl.ds(c * rows, rows)
            loads[c].wait()
            send_ref[csl, :] = xv_ref[csl, :].astype(jnp.bfloat16)
            rdma = pltpu.make_async_remote_copy(
                src_ref=send_ref.at[csl],
                dst_ref=recv_ref.at[csl],
                send_sem=s_sem1.at[c],
                recv_sem=r_sem1.at[c],
                device_id=y_nbr,
                device_id_type=pl.DeviceIdType.MESH,
            )
            rdma.start()
            phase1.append(rdma)

        phase2 = []
        stores = []
        for c in range(N_CHUNKS):
            phase1[c].wait_recv()
            csl = pl.ds(c * rows, rows)
            mine_ref[csl, :] = send_ref[csl, :] + recv_ref[csl, :]
            rdma = pltpu.make_async_remote_copy(
                src_ref=mine_ref.at[csl],
                dst_ref=oth_ref.at[csl],
                send_sem=s_sem2.at[c],
                recv_sem=r_sem2.at[c],
                device_id=x_nbr,
                device_id_type=pl.DeviceIdType.MESH,
            )
            rdma.start()
            phase2.append(rdma)
            st = pltpu.make_async_copy(
                mine_ref.at[csl],
                out_hbm.at[pl.ds(off + c * rows, rows)],
                st_my_sems.at[c],
            )
            st.start()
            stores.append(st)

        for c in range(N_CHUNKS):
            phase2[c].wait_recv()
            csl = pl.ds(c * rows, rows)
            st = pltpu.make_async_copy(
                oth_ref.at[csl],
                out_hbm.at[pl.ds(off_oth + c * rows, rows)],
                st_oth_sems.at[c],
            )
            st.start()
            stores.append(st)

        for st in stores:
            st.wait()
        for c in range(N_CHUNKS):
            phase1[c].wait_send()
            phase2[c].wait_send()

    return pl.pallas_call(
        body,
        out_shape=jax.ShapeDtypeStruct((m, n), jnp.bfloat16),
        in_specs=[pl.BlockSpec(memory_space=pltpu.MemorySpace.HBM)],
        out_specs=pl.BlockSpec(memory_space=pltpu.MemorySpace.HBM),
        scratch_shapes=[
            pltpu.VMEM((half, n), jnp.float32),
            pltpu.VMEM((half, n), jnp.bfloat16),
            pltpu.VMEM((half, n), jnp.bfloat16),
            pltpu.VMEM((half, n), jnp.bfloat16),
            pltpu.VMEM((half, n), jnp.bfloat16),
            pltpu.SemaphoreType.DMA((N_CHUNKS,)),
            pltpu.SemaphoreType.DMA((N_CHUNKS,)),
            pltpu.SemaphoreType.DMA((N_CHUNKS,)),
            pltpu.SemaphoreType.DMA((N_CHUNKS,)),
            pltpu.SemaphoreType.DMA((N_CHUNKS,)),
            pltpu.SemaphoreType.DMA((N_CHUNKS,)),
            pltpu.SemaphoreType.DMA((N_CHUNKS,)),
        ],
        compiler_params=pltpu.CompilerParams(
            collective_id=0, vmem_limit_bytes=100 * 1024 * 1024
        ),
    )(x)
